# baseline (speedup 1.0000x reference)
import functools

import jax
import jax.numpy as jnp
from jax.experimental import pallas as pl
from jax.experimental.pallas import tpu as pltpu


def _fused_kernel(scale_ref, x_ref, h_ref, w0_ref, b0_ref, w1_ref, b1_ref,
                  g_ref, b_ref, o_ref):
  f32 = jnp.float32
  d = w0_ref.shape[1]
  s = scale_ref[0, 0]

  w0t = w0_ref[0:d, :]
  w0b = w0_ref[d:2 * d, :]
  gamma = g_ref[...].reshape(1, d)
  w1 = w1_ref[...]
  w1c = (w1 - jnp.mean(w1, axis=1, keepdims=True)) * gamma
  b1r = b1_ref[...].reshape(1, d)
  b1c = (b1r - jnp.mean(b1r)) * gamma
  b0r = b0_ref[...].reshape(1, d)
  betar = b_ref[...].reshape(1, d)
  ig2 = 1.0 / (gamma * gamma)
  gmean = jnp.full((d, d), 1.0 / d, f32)

  pre = x_ref[...] * s + h_ref[...]
  up = pltpu.roll(pre, pre.shape[0] - 1, 0)
  dn = pltpu.roll(pre, 1, 0)
  row = jax.lax.broadcasted_iota(jnp.int32, pre.shape, 0)
  swapped = jnp.where(row % 2 == 0, up, dn)
  z = (jnp.dot(pre, w0t, preferred_element_type=f32)
       + jnp.dot(swapped, w0b, preferred_element_type=f32))
  y = jnp.maximum(z + b0r, 0.0)
  cg = jnp.dot(y, w1c, preferred_element_type=f32) + b1c
  var = jnp.dot(cg * cg * ig2, gmean, preferred_element_type=f32)
  o_ref[...] = (cg * jax.lax.rsqrt(var + 1e-5) + betar).astype(o_ref.dtype)


@functools.partial(jax.jit, static_argnames=("tile",))
def _gin_update(literal_embs, h, epsilon, w0, b0, w1, b1, ln_g, ln_b,
                tile=8192):
  n2, d = literal_embs.shape
  f32 = jnp.float32
  if n2 % tile != 0:
    tile = max(2, (n2 // 8) * 2)
  grid = pl.cdiv(n2, tile)
  scale = jnp.reshape(jnp.asarray(epsilon, f32) + 1.0, (1, 1))

  out = pl.pallas_call(
      _fused_kernel,
      out_shape=jax.ShapeDtypeStruct((n2, d), literal_embs.dtype),
      grid=(grid,),
      in_specs=[
          pl.BlockSpec(memory_space=pltpu.MemorySpace.SMEM),
          pl.BlockSpec((tile, d), lambda i: (i, 0)),
          pl.BlockSpec((tile, d), lambda i: (i, 0)),
          pl.BlockSpec((2 * d, d), lambda i: (0, 0)),
          pl.BlockSpec((d,), lambda i: (0,)),
          pl.BlockSpec((d, d), lambda i: (0, 0)),
          pl.BlockSpec((d,), lambda i: (0,)),
          pl.BlockSpec((d,), lambda i: (0,)),
          pl.BlockSpec((d,), lambda i: (0,)),
      ],
      out_specs=pl.BlockSpec((tile, d), lambda i: (i, 0)),
      compiler_params=pltpu.CompilerParams(
          dimension_semantics=("parallel",),
          vmem_limit_bytes=64 << 20),
  )(scale, literal_embs, h, w0.astype(f32), b0.astype(f32), w1.astype(f32),
    b1.astype(f32), ln_g.astype(f32), ln_b.astype(f32))
  return out


def kernel(literal_embs, h, epsilon, w0, b0, w1, b1, ln_g, ln_b):
  return _gin_update(literal_embs, h, epsilon, w0, b0, w1, b1, ln_g, ln_b)


# speedup vs baseline: 1.0236x; 1.0236x over previous
import functools

import jax
import jax.numpy as jnp
from jax.experimental import pallas as pl
from jax.experimental.pallas import tpu as pltpu


def _fused_kernel(scale_ref, x_ref, h_ref, w0_ref, b0_ref, w1_ref, b1_ref,
                  g_ref, b_ref, o_ref):
  f32 = jnp.float32
  d = w0_ref.shape[1]
  s = scale_ref[0, 0]

  w0t = w0_ref[0:d, :]
  w0b = w0_ref[d:2 * d, :]
  gamma = g_ref[...].reshape(1, d)
  w1 = w1_ref[...]
  w1c = (w1 - jnp.mean(w1, axis=1, keepdims=True)) * gamma
  b1r = b1_ref[...].reshape(1, d)
  b1c = (b1r - jnp.mean(b1r)) * gamma
  b0r = b0_ref[...].reshape(1, d)
  betar = b_ref[...].reshape(1, d)
  ig2 = 1.0 / (gamma * gamma)
  gmean = jnp.full((d, d), 1.0 / d, f32)

  pre = x_ref[...] * s + h_ref[...]
  up = pltpu.roll(pre, pre.shape[0] - 1, 0)
  dn = pltpu.roll(pre, 1, 0)
  row = jax.lax.broadcasted_iota(jnp.int32, pre.shape, 0)
  swapped = jnp.where(row % 2 == 0, up, dn)
  z = (jnp.dot(pre, w0t, preferred_element_type=f32)
       + jnp.dot(swapped, w0b, preferred_element_type=f32))
  y = jnp.maximum(z + b0r, 0.0)
  cg = jnp.dot(y, w1c, preferred_element_type=f32) + b1c
  var = jnp.dot(cg * cg * ig2, gmean, preferred_element_type=f32)
  o_ref[...] = (cg * jax.lax.rsqrt(var + 1e-5) + betar).astype(o_ref.dtype)


@functools.partial(jax.jit, static_argnames=("tile",))
def _gin_update(literal_embs, h, epsilon, w0, b0, w1, b1, ln_g, ln_b,
                tile=16384):
  n2, d = literal_embs.shape
  f32 = jnp.float32
  if n2 % tile != 0:
    tile = max(2, (n2 // 8) * 2)
  grid = pl.cdiv(n2, tile)
  scale = jnp.reshape(jnp.asarray(epsilon, f32) + 1.0, (1, 1))

  out = pl.pallas_call(
      _fused_kernel,
      out_shape=jax.ShapeDtypeStruct((n2, d), literal_embs.dtype),
      grid=(grid,),
      in_specs=[
          pl.BlockSpec(memory_space=pltpu.MemorySpace.SMEM),
          pl.BlockSpec((tile, d), lambda i: (i, 0)),
          pl.BlockSpec((tile, d), lambda i: (i, 0)),
          pl.BlockSpec((2 * d, d), lambda i: (0, 0)),
          pl.BlockSpec((d,), lambda i: (0,)),
          pl.BlockSpec((d, d), lambda i: (0, 0)),
          pl.BlockSpec((d,), lambda i: (0,)),
          pl.BlockSpec((d,), lambda i: (0,)),
          pl.BlockSpec((d,), lambda i: (0,)),
      ],
      out_specs=pl.BlockSpec((tile, d), lambda i: (i, 0)),
      compiler_params=pltpu.CompilerParams(
          dimension_semantics=("parallel",),
          vmem_limit_bytes=64 << 20),
  )(scale, literal_embs, h, w0.astype(f32), b0.astype(f32), w1.astype(f32),
    b1.astype(f32), ln_g.astype(f32), ln_b.astype(f32))
  return out


def kernel(literal_embs, h, epsilon, w0, b0, w1, b1, ln_g, ln_b):
  return _gin_update(literal_embs, h, epsilon, w0, b0, w1, b1, ln_g, ln_b)
